# baseline (speedup 1.0000x reference)
import jax
import jax.numpy as jnp
from jax import lax
from jax.experimental import pallas as pl
from jax.experimental.pallas import tpu as pltpu
from jax.experimental.pallas import tpu_sc as plsc

NCLS = 80
NCHAN = 85
GAMMA = 2.0
NTGT = 4096
CH = 128
CPS = NTGT // CH
NCHUNK = 3 * CPS
NSLOT = NCHUNK * CH
NC, NS = 2, 16
NTILE = NC * NS
WSCALE = (128, 64, 32)
HWS = tuple(w * w for w in WSCALE)
CELLS = tuple(16 * hw for hw in HWS)
CELL_OFF = (0, CELLS[0], CELLS[0] + CELLS[1])
NCELL = sum(CELLS)
DUMMY_CELL = NCELL
ZSPAN = 2048
ZPER = 11
MAPW = NS * ZPER * ZSPAN


def _sc_body(p0, p1, p2, idxh, cellh, outg,
             idx_v, cell_v, gbuf, zbuf, ones_v, cntmap, sem):
    cid = lax.axis_index("c")
    sid = lax.axis_index("s")
    wid = sid * NC + cid

    @pl.loop(0, ZSPAN // 16)
    def _(i):
        zbuf[pl.ds(i * 16, 16)] = jnp.zeros((16,), jnp.float32)

    @pl.loop(0, CH // 16)
    def _(i):
        ones_v[pl.ds(i * 16, 16)] = jnp.ones((16,), jnp.float32)

    @pl.loop(0, ZPER)
    def _(i):
        pltpu.sync_copy(zbuf, cntmap.at[pl.ds((sid * ZPER + i) * ZSPAN, ZSPAN)])

    plsc.subcore_barrier()

    @pl.loop(0, NCHUNK // NS)
    def _(k):
        jj = sid * (NCHUNK // NS) + k
        pltpu.sync_copy(cellh.at[jj], cell_v)
        pltpu.sync_copy(ones_v, cntmap.at[cell_v], add=True)

    plsc.subcore_barrier()

    for s_const, p in ((0, p0), (1, p1), (2, p2)):
        jj = wid + s_const * CPS
        pltpu.sync_copy(idxh.at[jj], idx_v)
        pltpu.sync_copy(cellh.at[jj], cell_v)
        for lo in range(0, NCHAN, 16):
            hi = min(lo + 16, NCHAN)

            @pl.loop(lo, hi)
            def _(c):
                pltpu.async_copy(p.at[idx_v.at[c]], gbuf.at[c], sem)

            @pl.loop(lo, hi)
            def _(c):
                pltpu.make_async_copy(p.at[idx_v.at[c]], gbuf.at[c],
                                      sem).wait()

        pltpu.async_copy(cntmap.at[cell_v], gbuf.at[NCHAN], sem).wait()
        pltpu.sync_copy(gbuf, outg.at[jj])


def _tc_body(pl0, pl1, pl2, g, par, obox, ocls, oobj, otot):
    i = pl.program_id(0)

    def bce0(x):
        return jnp.maximum(x, 0.0) + jnp.log(1.0 + jnp.exp(-jnp.abs(x)))

    @pl.when(i == 0)
    def _():
        cxv = par[0]
        cyv = par[1]
        bwv = par[2]
        bhv = par[3]
        wgt = par[4]
        clsf = par[5]
        gxf = par[6]
        gyf = par[7]
        wf = par[8]
        hf = par[9]
        valid = par[10]
        nobj = par[11]

        x0 = g[:, 0, :]
        x1 = g[:, 1, :]
        x2 = g[:, 2, :]
        x3 = g[:, 3, :]
        x4 = g[:, 4, :]
        mult = g[:, NCHAN, :]

        px = (1.0 / (1.0 + jnp.exp(-x0)) + gxf) / wf
        py = (1.0 / (1.0 + jnp.exp(-x1)) + gyf) / hf
        pw = jnp.exp(jnp.minimum(x2, 4.0)) / wf
        ph = jnp.exp(jnp.minimum(x3, 4.0)) / hf
        l1 = (jnp.abs(px - cxv) + jnp.abs(py - cyv)
              + jnp.abs(pw - bwv) + jnp.abs(ph - bhv)) * 0.25
        box_sum = jnp.sum(jnp.where(valid > 0, l1 * wgt, 0.0))

        xc = g[:, 5:5 + NCLS, :]
        lane_c = lax.broadcasted_iota(jnp.int32, (NCHUNK, NCLS, CH), 1)
        onehot = (lane_c.astype(jnp.float32) == clsf[:, None, :]).astype(jnp.float32)
        bcec = (jnp.maximum(xc, 0.0) - xc * onehot
                + jnp.log(1.0 + jnp.exp(-jnp.abs(xc))))
        cls_sum = jnp.sum(jnp.where(valid[:, None, :] > 0, bcec, 0.0)) / NCLS

        corr = jnp.sum(jnp.where(valid > 0, x4 / mult * nobj, 0.0))

        d1 = jnp.sum(bce0(pl1[:, 0, :, :])) / (16.0 * HWS[1])
        d2 = jnp.sum(bce0(pl2[:, 0, :, :])) / (16.0 * HWS[2])

        obox[0, 0] = box_sum / NTGT
        ocls[0, 0] = cls_sum / NTGT
        oobj[0, 0] = d1 + d2 - corr

    oobj[0, 0] += jnp.sum(bce0(pl0[0, 0])) / (16.0 * HWS[0])

    @pl.when(i == 15)
    def _():
        otot[0, 0] = obox[0, 0] + ocls[0, 0] + oobj[0, 0]


def kernel(preds_0, preds_1, preds_2, targets):
    t = lax.stop_gradient(targets)
    b = t[:, 0].astype(jnp.int32)
    clsf = t[:, 1]
    cx, cy, bw, bh = t[:, 2], t[:, 3], t[:, 4], t[:, 5]
    area = jnp.maximum(bw * bh, 1e-6)
    sidx = jnp.where(area <= 0.01, 0, jnp.where(area <= 0.03, 1, 2)).astype(jnp.int32)
    weight = 1.0 + GAMMA * (1.0 - jnp.sqrt(area))

    ws = jnp.array(WSCALE, jnp.int32)[sidx]
    wsf = ws.astype(jnp.float32)
    gx = jnp.clip((cx * wsf).astype(jnp.int32), 0, ws - 1)
    gy = jnp.clip((cy * wsf).astype(jnp.int32), 0, ws - 1)
    hw = jnp.array(HWS, jnp.int32)[sidx]
    base = (b * NCHAN * ws + gy) * ws + gx
    cell = jnp.array(CELL_OFF, jnp.int32)[sidx] + (b * ws + gy) * ws + gx

    order = jnp.argsort(sidx, stable=True)
    ssorted = sidx[order]
    cnt = jnp.stack([jnp.sum(sidx == s) for s in range(3)]).astype(jnp.int32)
    start_in_sorted = jnp.concatenate(
        [jnp.zeros((1,), jnp.int32), jnp.cumsum(cnt)[:2].astype(jnp.int32)])
    rank = jnp.arange(NTGT, dtype=jnp.int32) - start_in_sorted[ssorted]
    slot = ssorted * NTGT + rank

    def scat(vals, fill):
        a = jnp.full((NSLOT,), fill, vals.dtype)
        return a.at[slot].set(vals[order])

    base_s = scat(base, 0)
    cell_s = scat(cell, DUMMY_CELL)
    hw_s = scat(hw, 1)
    idx_all = (base_s.reshape(NCHUNK, 1, CH)
               + jnp.arange(NCHAN, dtype=jnp.int32).reshape(1, NCHAN, 1)
               * hw_s.reshape(NCHUNK, 1, CH))
    cells_arr = cell_s.reshape(NCHUNK, CH)

    f32 = jnp.float32
    par = jnp.stack([
        scat(cx, 0.0), scat(cy, 0.0), scat(bw, 0.0), scat(bh, 0.0),
        scat(weight, 0.0), scat(clsf, 0.0),
        scat(gx.astype(f32), 0.0), scat(gy.astype(f32), 0.0),
        scat(wsf, 1.0), scat(wsf, 1.0),
        scat(jnp.ones((NTGT,), f32), 0.0),
        scat(1.0 / (16.0 * hw.astype(f32)), 0.0),
    ]).reshape(12, NCHUNK, CH)

    sc = pl.kernel(
        _sc_body,
        out_type=jax.ShapeDtypeStruct((NCHUNK, 88, CH), f32),
        mesh=plsc.VectorSubcoreMesh(core_axis_name="c", subcore_axis_name="s"),
        scratch_types=[
            pltpu.VMEM((NCHAN, CH), jnp.int32),
            pltpu.VMEM((CH,), jnp.int32),
            pltpu.VMEM((88, CH), f32),
            pltpu.VMEM((ZSPAN,), f32),
            pltpu.VMEM((CH,), f32),
            pltpu.VMEM_SHARED((MAPW,), f32),
            pltpu.SemaphoreType.DMA,
        ],
    )
    g = sc(preds_0.reshape(-1), preds_1.reshape(-1), preds_2.reshape(-1),
           idx_all, cells_arr)

    losses = pl.pallas_call(
        _tc_body,
        grid=(16,),
        in_specs=[
            pl.BlockSpec((1, 1, 128, 128), lambda i: (i, 4, 0, 0)),
            pl.BlockSpec((16, 1, 64, 64), lambda i: (0, 4, 0, 0)),
            pl.BlockSpec((16, 1, 32, 32), lambda i: (0, 4, 0, 0)),
            pl.BlockSpec((NCHUNK, 88, CH), lambda i: (0, 0, 0)),
            pl.BlockSpec((12, NCHUNK, CH), lambda i: (0, 0, 0)),
        ],
        out_specs=[pl.BlockSpec((1, 1), lambda i: (0, 0),
                                memory_space=pltpu.SMEM)] * 4,
        out_shape=[jax.ShapeDtypeStruct((1, 1), f32)] * 4,
    )(preds_0, preds_1, preds_2, g, par)
    obox, ocls, oobj, otot = losses
    return otot[0, 0], obox[0, 0], oobj[0, 0], ocls[0, 0]


# speedup vs baseline: 8.4246x; 8.4246x over previous
import jax
import jax.numpy as jnp
from jax import lax
from jax.experimental import pallas as pl
from jax.experimental.pallas import tpu as pltpu
from jax.experimental.pallas import tpu_sc as plsc

NCLS = 80
NCHAN = 85
GAMMA = 2.0
NTGT = 4096
CH = 128
CPS = NTGT // CH
NCHUNK = 3 * CPS
NSLOT = NCHUNK * CH
NC, NS = 2, 16
NTILE = NC * NS
WSCALE = (128, 64, 32)
HWS = tuple(w * w for w in WSCALE)
CELLS = tuple(16 * hw for hw in HWS)
CELL_OFF = (0, CELLS[0], CELLS[0] + CELLS[1])
NCELL = sum(CELLS)
DUMMY_CELL = NCELL
ZSPAN = 2048
ZPER = 11
MAPW = NS * ZPER * ZSPAN


def _sc_body(p0, p1, p2, idxh, cellh, outg,
             idx_v, cell_v, gbuf, zbuf, ones_v, cntmap, sem):
    cid = lax.axis_index("c")
    sid = lax.axis_index("s")
    wid = sid * NC + cid

    @pl.loop(0, ZSPAN // 16)
    def _(i):
        zbuf[pl.ds(i * 16, 16)] = jnp.zeros((16,), jnp.float32)

    @pl.loop(0, CH // 16)
    def _(i):
        ones_v[pl.ds(i * 16, 16)] = jnp.ones((16,), jnp.float32)

    @pl.loop(0, ZPER)
    def _(i):
        pltpu.sync_copy(zbuf, cntmap.at[pl.ds((sid * ZPER + i) * ZSPAN, ZSPAN)])

    plsc.subcore_barrier()

    @pl.loop(0, NCHUNK // NS)
    def _(k):
        jj = sid * (NCHUNK // NS) + k
        pltpu.sync_copy(cellh.at[jj], cell_v)
        pltpu.sync_copy(ones_v, cntmap.at[cell_v], add=True)

    plsc.subcore_barrier()

    for s_const, p in ((0, p0), (1, p1), (2, p2)):
        jj = wid + s_const * CPS
        pltpu.sync_copy(idxh.at[jj], idx_v)
        pltpu.sync_copy(cellh.at[jj], cell_v)
        for lo in range(0, NCHAN, 16):
            hi = min(lo + 16, NCHAN)

            @pl.loop(lo, hi)
            def _(c):
                pltpu.async_copy(p.at[idx_v.at[c]], gbuf.at[c], sem)

            @pl.loop(lo, hi)
            def _(c):
                pltpu.make_async_copy(p.at[idx_v.at[c]], gbuf.at[c],
                                      sem).wait()

        pltpu.async_copy(cntmap.at[cell_v], gbuf.at[NCHAN], sem).wait()
        pltpu.sync_copy(gbuf, outg.at[jj])


def _tc_body(pl0, pl1, pl2, g, par, obox, ocls, oobj, otot):
    i = pl.program_id(0)

    def bce0(x):
        return jnp.maximum(x, 0.0) + jnp.log(1.0 + jnp.exp(-jnp.abs(x)))

    @pl.when(i == 0)
    def _():
        cxv = par[0]
        cyv = par[1]
        bwv = par[2]
        bhv = par[3]
        wgt = par[4]
        clsf = par[5]
        gxf = par[6]
        gyf = par[7]
        wf = par[8]
        hf = par[9]
        valid = par[10]
        nobj = par[11]

        x0 = g[:, 0, :]
        x1 = g[:, 1, :]
        x2 = g[:, 2, :]
        x3 = g[:, 3, :]
        x4 = g[:, 4, :]
        mult = g[:, NCHAN, :]

        px = (1.0 / (1.0 + jnp.exp(-x0)) + gxf) / wf
        py = (1.0 / (1.0 + jnp.exp(-x1)) + gyf) / hf
        pw = jnp.exp(jnp.minimum(x2, 4.0)) / wf
        ph = jnp.exp(jnp.minimum(x3, 4.0)) / hf
        l1 = (jnp.abs(px - cxv) + jnp.abs(py - cyv)
              + jnp.abs(pw - bwv) + jnp.abs(ph - bhv)) * 0.25
        box_sum = jnp.sum(jnp.where(valid > 0, l1 * wgt, 0.0))

        xc = g[:, 5:5 + NCLS, :]
        lane_c = lax.broadcasted_iota(jnp.int32, (NCHUNK, NCLS, CH), 1)
        onehot = (lane_c.astype(jnp.float32) == clsf[:, None, :]).astype(jnp.float32)
        bcec = (jnp.maximum(xc, 0.0) - xc * onehot
                + jnp.log(1.0 + jnp.exp(-jnp.abs(xc))))
        cls_sum = jnp.sum(jnp.where(valid[:, None, :] > 0, bcec, 0.0)) / NCLS

        corr = jnp.sum(jnp.where(valid > 0, x4 / mult * nobj, 0.0))

        d1 = jnp.sum(bce0(pl1[:, 0, :, :])) / (16.0 * HWS[1])
        d2 = jnp.sum(bce0(pl2[:, 0, :, :])) / (16.0 * HWS[2])

        obox[0, 0] = box_sum / NTGT
        ocls[0, 0] = cls_sum / NTGT
        oobj[0, 0] = d1 + d2 - corr

    oobj[0, 0] += jnp.sum(bce0(pl0[0, 0])) / (16.0 * HWS[0])

    @pl.when(i == 15)
    def _():
        otot[0, 0] = obox[0, 0] + ocls[0, 0] + oobj[0, 0]


def kernel(preds_0, preds_1, preds_2, targets):
    t = lax.stop_gradient(targets)
    b = t[:, 0].astype(jnp.int32)
    clsf = t[:, 1]
    cx, cy, bw, bh = t[:, 2], t[:, 3], t[:, 4], t[:, 5]
    area = jnp.maximum(bw * bh, 1e-6)
    sidx = jnp.where(area <= 0.01, 0, jnp.where(area <= 0.03, 1, 2)).astype(jnp.int32)
    weight = 1.0 + GAMMA * (1.0 - jnp.sqrt(area))

    ws = jnp.array(WSCALE, jnp.int32)[sidx]
    wsf = ws.astype(jnp.float32)
    gx = jnp.clip((cx * wsf).astype(jnp.int32), 0, ws - 1)
    gy = jnp.clip((cy * wsf).astype(jnp.int32), 0, ws - 1)
    hw = jnp.array(HWS, jnp.int32)[sidx]
    base = (b * NCHAN * ws + gy) * ws + gx
    cell = jnp.array(CELL_OFF, jnp.int32)[sidx] + (b * ws + gy) * ws + gx

    order = jnp.argsort(sidx, stable=True)
    ssorted = sidx[order]
    cnt = jnp.stack([jnp.sum(sidx == s) for s in range(3)]).astype(jnp.int32)
    start_in_sorted = jnp.concatenate(
        [jnp.zeros((1,), jnp.int32), jnp.cumsum(cnt)[:2].astype(jnp.int32)])
    rank = jnp.arange(NTGT, dtype=jnp.int32) - start_in_sorted[ssorted]
    slot = ssorted * NTGT + rank

    def scat(vals, fill):
        a = jnp.full((NSLOT,), fill, vals.dtype)
        return a.at[slot].set(vals[order])

    def scat_arr(vals, fill_arr):
        return fill_arr.at[slot].set(vals[order])

    gid = jnp.arange(NSLOT, dtype=jnp.int32)
    scale_of = gid // (CPS * CH)
    hw_of = jnp.array(HWS, jnp.int32)[scale_of]
    dummy_base = (gid % 16) * NCHAN * hw_of + (gid * 61) % hw_of
    dummy_cell = NCELL + gid % (MAPW - NCELL - 8)

    base_s = scat_arr(base, dummy_base)
    cell_s = scat_arr(cell, dummy_cell)
    hw_s = scat_arr(hw, hw_of)
    idx_all = (base_s.reshape(NCHUNK, 1, CH)
               + jnp.arange(NCHAN, dtype=jnp.int32).reshape(1, NCHAN, 1)
               * hw_s.reshape(NCHUNK, 1, CH))
    cells_arr = cell_s.reshape(NCHUNK, CH)

    f32 = jnp.float32
    par = jnp.stack([
        scat(cx, 0.0), scat(cy, 0.0), scat(bw, 0.0), scat(bh, 0.0),
        scat(weight, 0.0), scat(clsf, 0.0),
        scat(gx.astype(f32), 0.0), scat(gy.astype(f32), 0.0),
        scat(wsf, 1.0), scat(wsf, 1.0),
        scat(jnp.ones((NTGT,), f32), 0.0),
        scat(1.0 / (16.0 * hw.astype(f32)), 0.0),
    ]).reshape(12, NCHUNK, CH)

    sc = pl.kernel(
        _sc_body,
        out_type=jax.ShapeDtypeStruct((NCHUNK, 88, CH), f32),
        mesh=plsc.VectorSubcoreMesh(core_axis_name="c", subcore_axis_name="s"),
        scratch_types=[
            pltpu.VMEM((NCHAN, CH), jnp.int32),
            pltpu.VMEM((CH,), jnp.int32),
            pltpu.VMEM((88, CH), f32),
            pltpu.VMEM((ZSPAN,), f32),
            pltpu.VMEM((CH,), f32),
            pltpu.VMEM_SHARED((MAPW,), f32),
            pltpu.SemaphoreType.DMA,
        ],
    )
    g = sc(preds_0.reshape(-1), preds_1.reshape(-1), preds_2.reshape(-1),
           idx_all, cells_arr)

    losses = pl.pallas_call(
        _tc_body,
        grid=(16,),
        in_specs=[
            pl.BlockSpec((1, 1, 128, 128), lambda i: (i, 4, 0, 0)),
            pl.BlockSpec((16, 1, 64, 64), lambda i: (0, 4, 0, 0)),
            pl.BlockSpec((16, 1, 32, 32), lambda i: (0, 4, 0, 0)),
            pl.BlockSpec((NCHUNK, 88, CH), lambda i: (0, 0, 0)),
            pl.BlockSpec((12, NCHUNK, CH), lambda i: (0, 0, 0)),
        ],
        out_specs=[pl.BlockSpec((1, 1), lambda i: (0, 0),
                                memory_space=pltpu.SMEM)] * 4,
        out_shape=[jax.ShapeDtypeStruct((1, 1), f32)] * 4,
    )(preds_0, preds_1, preds_2, g, par)
    obox, ocls, oobj, otot = losses
    return otot[0, 0], obox[0, 0], oobj[0, 0], ocls[0, 0]


# speedup vs baseline: 15.5516x; 1.8460x over previous
import jax
import jax.numpy as jnp
from jax import lax
from jax.experimental import pallas as pl
from jax.experimental.pallas import tpu as pltpu
from jax.experimental.pallas import tpu_sc as plsc

NCLS = 80
NCHAN = 85
GAMMA = 2.0
NTGT = 4096
CH = 128
CPS = NTGT // CH
NCHUNK = 3 * CPS
NSLOT = NCHUNK * CH
NC, NS = 2, 16
NTILE = NC * NS
WSCALE = (128, 64, 32)
HWS = tuple(w * w for w in WSCALE)
CELLS = tuple(16 * hw for hw in HWS)
CELL_OFF = (0, CELLS[0], CELLS[0] + CELLS[1])
NCELL = sum(CELLS)
ZSPAN = 2048
ZPER = 11
MAPW = NS * ZPER * ZSPAN
WAVE = 16


def _sc_body(p0, p1, p2, idxh, cellh, outg,
             idx_v, cell_v, gbuf, zbuf, ones_v, cntmap, sem):
    cid = lax.axis_index("c")
    sid = lax.axis_index("s")
    wid = sid * NC + cid

    @pl.loop(0, ZSPAN // 16)
    def _(i):
        zbuf[pl.ds(i * 16, 16)] = jnp.zeros((16,), jnp.float32)

    @pl.loop(0, CH // 16)
    def _(i):
        ones_v[pl.ds(i * 16, 16)] = jnp.ones((16,), jnp.float32)

    @pl.loop(0, ZPER)
    def _(i):
        pltpu.sync_copy(zbuf, cntmap.at[pl.ds((sid * ZPER + i) * ZSPAN, ZSPAN)])

    plsc.subcore_barrier()

    @pl.loop(0, NCHUNK // NS)
    def _(k):
        jj = sid * (NCHUNK // NS) + k
        pltpu.sync_copy(cellh.at[jj], cell_v)
        pltpu.sync_copy(ones_v, cntmap.at[cell_v], add=True)

    plsc.subcore_barrier()

    for s_const, p in ((0, p0), (1, p1), (2, p2)):
        jj = wid + s_const * CPS
        pltpu.sync_copy(idxh.at[jj], idx_v)
        pltpu.sync_copy(cellh.at[jj], cell_v)
        for lo in range(0, NCHAN, WAVE):
            w = min(lo + WAVE, NCHAN) - lo

            @pl.loop(lo, lo + w)
            def _(c):
                pltpu.async_copy(p.at[idx_v.at[c]], gbuf.at[c], sem)

            pltpu.make_async_copy(p.at[pl.ds(0, w * CH)],
                                  zbuf.at[pl.ds(0, w * CH)], sem).wait()

        pltpu.async_copy(cntmap.at[cell_v], gbuf.at[NCHAN], sem).wait()
        pltpu.sync_copy(gbuf, outg.at[jj])


def _tc_body(pl0, pl1, pl2, g, par, obox, ocls, oobj, otot):
    i = pl.program_id(0)

    def bce0(x):
        return jnp.maximum(x, 0.0) + jnp.log(1.0 + jnp.exp(-jnp.abs(x)))

    @pl.when(i == 0)
    def _():
        cxv = par[0]
        cyv = par[1]
        bwv = par[2]
        bhv = par[3]
        wgt = par[4]
        clsf = par[5]
        gxf = par[6]
        gyf = par[7]
        wf = par[8]
        hf = par[9]
        valid = par[10]
        nobj = par[11]

        x0 = g[:, 0, :]
        x1 = g[:, 1, :]
        x2 = g[:, 2, :]
        x3 = g[:, 3, :]
        x4 = g[:, 4, :]
        mult = g[:, NCHAN, :]

        px = (1.0 / (1.0 + jnp.exp(-x0)) + gxf) / wf
        py = (1.0 / (1.0 + jnp.exp(-x1)) + gyf) / hf
        pw = jnp.exp(jnp.minimum(x2, 4.0)) / wf
        ph = jnp.exp(jnp.minimum(x3, 4.0)) / hf
        l1 = (jnp.abs(px - cxv) + jnp.abs(py - cyv)
              + jnp.abs(pw - bwv) + jnp.abs(ph - bhv)) * 0.25
        box_sum = jnp.sum(jnp.where(valid > 0, l1 * wgt, 0.0))

        xc = g[:, 5:5 + NCLS, :]
        lane_c = lax.broadcasted_iota(jnp.int32, (NCHUNK, NCLS, CH), 1)
        onehot = (lane_c.astype(jnp.float32) == clsf[:, None, :]).astype(jnp.float32)
        bcec = (jnp.maximum(xc, 0.0) - xc * onehot
                + jnp.log(1.0 + jnp.exp(-jnp.abs(xc))))
        cls_sum = jnp.sum(jnp.where(valid[:, None, :] > 0, bcec, 0.0)) / NCLS

        corr = jnp.sum(jnp.where(valid > 0, x4 / mult * nobj, 0.0))

        d1 = jnp.sum(bce0(pl1[:, 0, :, :])) / (16.0 * HWS[1])
        d2 = jnp.sum(bce0(pl2[:, 0, :, :])) / (16.0 * HWS[2])

        obox[0, 0] = box_sum / NTGT
        ocls[0, 0] = cls_sum / NTGT
        oobj[0, 0] = d1 + d2 - corr

    oobj[0, 0] += jnp.sum(bce0(pl0[0, 0])) / (16.0 * HWS[0])

    @pl.when(i == 15)
    def _():
        otot[0, 0] = obox[0, 0] + ocls[0, 0] + oobj[0, 0]


def kernel(preds_0, preds_1, preds_2, targets):
    f32 = jnp.float32
    t = lax.stop_gradient(targets)
    b = t[:, 0].astype(jnp.int32)
    clsv = t[:, 1]
    cx, cy, bw, bh = t[:, 2], t[:, 3], t[:, 4], t[:, 5]
    area = jnp.maximum(bw * bh, 1e-6)
    sidx = jnp.where(area <= 0.01, 0, jnp.where(area <= 0.03, 1, 2)).astype(jnp.int32)
    weight = 1.0 + GAMMA * (1.0 - jnp.sqrt(area))

    wsa = jnp.array(WSCALE, jnp.int32).reshape(3, 1)
    hwa = jnp.array(HWS, jnp.int32).reshape(3, 1)
    offa = jnp.array(CELL_OFF, jnp.int32).reshape(3, 1)
    wsf3 = wsa.astype(f32)
    gx3 = jnp.clip((cx[None] * wsf3).astype(jnp.int32), 0, wsa - 1)
    gy3 = jnp.clip((cy[None] * wsf3).astype(jnp.int32), 0, wsa - 1)
    base3 = (b[None] * NCHAN * wsa + gy3) * wsa + gx3
    cell3 = offa + (b[None] * wsa + gy3) * wsa + gx3
    mask3 = sidx[None] == jnp.arange(3, dtype=jnp.int32)[:, None]

    gid = jnp.arange(3 * NTGT, dtype=jnp.int32).reshape(3, NTGT)
    cell3 = jnp.where(mask3, cell3, NCELL + gid % (MAPW - NCELL - 8))

    hw3 = jnp.broadcast_to(hwa, (3, NTGT))
    idx_all = (base3.reshape(NCHUNK, 1, CH)
               + jnp.arange(NCHAN, dtype=jnp.int32).reshape(1, NCHAN, 1)
               * hw3.reshape(NCHUNK, 1, CH))
    cells_arr = cell3.reshape(NCHUNK, CH)

    maskf = mask3.astype(f32)

    def brd(v):
        return jnp.broadcast_to(v[None], (3, NTGT))

    par = jnp.stack([
        brd(cx), brd(cy), brd(bw), brd(bh), brd(weight), brd(clsv),
        gx3.astype(f32), gy3.astype(f32),
        jnp.broadcast_to(wsf3, (3, NTGT)), jnp.broadcast_to(wsf3, (3, NTGT)),
        maskf,
        maskf / (16.0 * hwa.astype(f32)),
    ]).reshape(12, NCHUNK, CH)

    sc = pl.kernel(
        _sc_body,
        out_type=jax.ShapeDtypeStruct((NCHUNK, 88, CH), f32),
        mesh=plsc.VectorSubcoreMesh(core_axis_name="c", subcore_axis_name="s"),
        scratch_types=[
            pltpu.VMEM((NCHAN, CH), jnp.int32),
            pltpu.VMEM((CH,), jnp.int32),
            pltpu.VMEM((88, CH), f32),
            pltpu.VMEM((ZSPAN,), f32),
            pltpu.VMEM((CH,), f32),
            pltpu.VMEM_SHARED((MAPW,), f32),
            pltpu.SemaphoreType.DMA,
        ],
    )
    g = sc(preds_0.reshape(-1), preds_1.reshape(-1), preds_2.reshape(-1),
           idx_all, cells_arr)

    losses = pl.pallas_call(
        _tc_body,
        grid=(16,),
        in_specs=[
            pl.BlockSpec((1, 1, 128, 128), lambda i: (i, 4, 0, 0)),
            pl.BlockSpec((16, 1, 64, 64), lambda i: (0, 4, 0, 0)),
            pl.BlockSpec((16, 1, 32, 32), lambda i: (0, 4, 0, 0)),
            pl.BlockSpec((NCHUNK, 88, CH), lambda i: (0, 0, 0)),
            pl.BlockSpec((12, NCHUNK, CH), lambda i: (0, 0, 0)),
        ],
        out_specs=[pl.BlockSpec((1, 1), lambda i: (0, 0),
                                memory_space=pltpu.SMEM)] * 4,
        out_shape=[jax.ShapeDtypeStruct((1, 1), f32)] * 4,
    )(preds_0, preds_1, preds_2, g, par)
    obox, ocls, oobj, otot = losses
    return otot[0, 0], obox[0, 0], oobj[0, 0], ocls[0, 0]


# speedup vs baseline: 16.7706x; 1.0784x over previous
import jax
import jax.numpy as jnp
from jax import lax
from jax.experimental import pallas as pl
from jax.experimental.pallas import tpu as pltpu
from jax.experimental.pallas import tpu_sc as plsc

NCLS = 80
NCHAN = 85
GAMMA = 2.0
NTGT = 4096
CH = 128
CPS = NTGT // CH
NCHUNK = 3 * CPS
NSLOT = NCHUNK * CH
NC, NS = 2, 16
NTILE = NC * NS
WSCALE = (128, 64, 32)
HWS = tuple(w * w for w in WSCALE)
CELLS = tuple(16 * hw for hw in HWS)
CELL_OFF = (0, CELLS[0], CELLS[0] + CELLS[1])
NCELL = sum(CELLS)
ZSPAN = 2048
ZPER = 11
MAPW = NS * ZPER * ZSPAN
WAVE = 17


def _sc_body(p0, p1, p2, idxh, cellh, outg,
             idx2, cell2, cellS, gbuf2, zbuf, dbuf, ones_v, cntmap,
             semg, semz, semm):
    cid = lax.axis_index("c")
    sid = lax.axis_index("s")
    wid = sid * NC + cid
    tabs = (p0, p1, p2)

    @pl.loop(0, ZSPAN // 16)
    def _(i):
        zbuf[pl.ds(i * 16, 16)] = jnp.zeros((16,), jnp.float32)

    @pl.loop(0, CH // 16)
    def _(i):
        ones_v[pl.ds(i * 16, 16)] = jnp.ones((16,), jnp.float32)

    def fire(s_const, buf):
        jj = wid + s_const * CPS
        pltpu.sync_copy(idxh.at[jj], idx2.at[buf])
        pltpu.sync_copy(cellh.at[jj], cell2.at[buf])
        p = tabs[s_const]

        @pl.loop(0, NCHAN)
        def _(c):
            pltpu.async_copy(p.at[idx2.at[buf, c]], gbuf2.at[buf, c], semg)

    def drain(s_const):
        pltpu.make_async_copy(tabs[s_const].at[pl.ds(0, NCHAN * CH)],
                              dbuf.at[pl.ds(0, NCHAN * CH)], semg).wait()

    def finish(s_const, buf):
        jj = wid + s_const * CPS
        pltpu.async_copy(cntmap.at[cell2.at[buf]], gbuf2.at[buf, NCHAN],
                         semm).wait()
        pltpu.sync_copy(gbuf2.at[buf], outg.at[jj])

    fire(0, 0)

    @pl.loop(0, ZPER)
    def _(i):
        pltpu.async_copy(zbuf,
                         cntmap.at[pl.ds((sid * ZPER + i) * ZSPAN, ZSPAN)],
                         semz)
    pltpu.make_async_copy(p0.at[pl.ds(0, ZPER * ZSPAN)], dbuf, semz).wait()
    plsc.subcore_barrier()

    @pl.loop(0, NCHUNK // NS)
    def _(k):
        jj = sid * (NCHUNK // NS) + k
        pltpu.sync_copy(cellh.at[jj], cellS)
        pltpu.sync_copy(ones_v, cntmap.at[cellS], add=True)
    plsc.subcore_barrier()

    drain(0)
    fire(1, 1)
    finish(0, 0)
    drain(1)
    fire(2, 0)
    finish(1, 1)
    drain(2)
    finish(2, 0)


def _tc_body(pl0, pl1, pl2, g, par, obox, ocls, oobj, otot):
    i = pl.program_id(0)

    def bce0(x):
        return jnp.maximum(x, 0.0) + jnp.log(1.0 + jnp.exp(-jnp.abs(x)))

    @pl.when(i == 0)
    def _():
        cxv = par[0]
        cyv = par[1]
        bwv = par[2]
        bhv = par[3]
        wgt = par[4]
        clsf = par[5]
        gxf = par[6]
        gyf = par[7]
        wf = par[8]
        hf = par[9]
        valid = par[10]
        nobj = par[11]

        x0 = g[:, 0, :]
        x1 = g[:, 1, :]
        x2 = g[:, 2, :]
        x3 = g[:, 3, :]
        x4 = g[:, 4, :]
        mult = g[:, NCHAN, :]

        px = (1.0 / (1.0 + jnp.exp(-x0)) + gxf) / wf
        py = (1.0 / (1.0 + jnp.exp(-x1)) + gyf) / hf
        pw = jnp.exp(jnp.minimum(x2, 4.0)) / wf
        ph = jnp.exp(jnp.minimum(x3, 4.0)) / hf
        l1 = (jnp.abs(px - cxv) + jnp.abs(py - cyv)
              + jnp.abs(pw - bwv) + jnp.abs(ph - bhv)) * 0.25
        box_sum = jnp.sum(jnp.where(valid > 0, l1 * wgt, 0.0))

        xc = g[:, 5:5 + NCLS, :]
        lane_c = lax.broadcasted_iota(jnp.int32, (NCHUNK, NCLS, CH), 1)
        onehot = (lane_c.astype(jnp.float32) == clsf[:, None, :]).astype(jnp.float32)
        bcec = (jnp.maximum(xc, 0.0) - xc * onehot
                + jnp.log(1.0 + jnp.exp(-jnp.abs(xc))))
        cls_sum = jnp.sum(jnp.where(valid[:, None, :] > 0, bcec, 0.0)) / NCLS

        corr = jnp.sum(jnp.where(valid > 0, x4 / mult * nobj, 0.0))

        d1 = jnp.sum(bce0(pl1[:, 0, :, :])) / (16.0 * HWS[1])
        d2 = jnp.sum(bce0(pl2[:, 0, :, :])) / (16.0 * HWS[2])

        obox[0, 0] = box_sum / NTGT
        ocls[0, 0] = cls_sum / NTGT
        oobj[0, 0] = d1 + d2 - corr

    oobj[0, 0] += jnp.sum(bce0(pl0[0, 0])) / (16.0 * HWS[0])

    @pl.when(i == 15)
    def _():
        otot[0, 0] = obox[0, 0] + ocls[0, 0] + oobj[0, 0]


def kernel(preds_0, preds_1, preds_2, targets):
    f32 = jnp.float32
    t = lax.stop_gradient(targets)
    b = t[:, 0].astype(jnp.int32)
    clsv = t[:, 1]
    cx, cy, bw, bh = t[:, 2], t[:, 3], t[:, 4], t[:, 5]
    area = jnp.maximum(bw * bh, 1e-6)
    sidx = jnp.where(area <= 0.01, 0, jnp.where(area <= 0.03, 1, 2)).astype(jnp.int32)
    weight = 1.0 + GAMMA * (1.0 - jnp.sqrt(area))

    wsa = jnp.array(WSCALE, jnp.int32).reshape(3, 1)
    hwa = jnp.array(HWS, jnp.int32).reshape(3, 1)
    offa = jnp.array(CELL_OFF, jnp.int32).reshape(3, 1)
    wsf3 = wsa.astype(f32)
    gx3 = jnp.clip((cx[None] * wsf3).astype(jnp.int32), 0, wsa - 1)
    gy3 = jnp.clip((cy[None] * wsf3).astype(jnp.int32), 0, wsa - 1)
    base3 = (b[None] * NCHAN * wsa + gy3) * wsa + gx3
    cell3 = offa + (b[None] * wsa + gy3) * wsa + gx3
    mask3 = sidx[None] == jnp.arange(3, dtype=jnp.int32)[:, None]

    gid = jnp.arange(3 * NTGT, dtype=jnp.int32).reshape(3, NTGT)
    cell3 = jnp.where(mask3, cell3, NCELL + gid % (MAPW - NCELL - 8))

    hw3 = jnp.broadcast_to(hwa, (3, NTGT))
    idx_all = (base3.reshape(NCHUNK, 1, CH)
               + jnp.arange(NCHAN, dtype=jnp.int32).reshape(1, NCHAN, 1)
               * hw3.reshape(NCHUNK, 1, CH))
    cells_arr = cell3.reshape(NCHUNK, CH)

    maskf = mask3.astype(f32)

    def brd(v):
        return jnp.broadcast_to(v[None], (3, NTGT))

    par = jnp.stack([
        brd(cx), brd(cy), brd(bw), brd(bh), brd(weight), brd(clsv),
        gx3.astype(f32), gy3.astype(f32),
        jnp.broadcast_to(wsf3, (3, NTGT)), jnp.broadcast_to(wsf3, (3, NTGT)),
        maskf,
        maskf / (16.0 * hwa.astype(f32)),
    ]).reshape(12, NCHUNK, CH)

    sc = pl.kernel(
        _sc_body,
        out_type=jax.ShapeDtypeStruct((NCHUNK, 88, CH), f32),
        mesh=plsc.VectorSubcoreMesh(core_axis_name="c", subcore_axis_name="s"),
        scratch_types=[
            pltpu.VMEM((2, NCHAN, CH), jnp.int32),
            pltpu.VMEM((2, CH), jnp.int32),
            pltpu.VMEM((CH,), jnp.int32),
            pltpu.VMEM((2, 88, CH), f32),
            pltpu.VMEM((ZSPAN,), f32),
            pltpu.VMEM((ZPER * ZSPAN,), f32),
            pltpu.VMEM((CH,), f32),
            pltpu.VMEM_SHARED((MAPW,), f32),
            pltpu.SemaphoreType.DMA,
            pltpu.SemaphoreType.DMA,
            pltpu.SemaphoreType.DMA,
        ],
    )
    g = sc(preds_0.reshape(-1), preds_1.reshape(-1), preds_2.reshape(-1),
           idx_all, cells_arr)

    losses = pl.pallas_call(
        _tc_body,
        grid=(16,),
        in_specs=[
            pl.BlockSpec((1, 1, 128, 128), lambda i: (i, 4, 0, 0)),
            pl.BlockSpec((16, 1, 64, 64), lambda i: (0, 4, 0, 0)),
            pl.BlockSpec((16, 1, 32, 32), lambda i: (0, 4, 0, 0)),
            pl.BlockSpec((NCHUNK, 88, CH), lambda i: (0, 0, 0)),
            pl.BlockSpec((12, NCHUNK, CH), lambda i: (0, 0, 0)),
        ],
        out_specs=[pl.BlockSpec((1, 1), lambda i: (0, 0),
                                memory_space=pltpu.SMEM)] * 4,
        out_shape=[jax.ShapeDtypeStruct((1, 1), f32)] * 4,
    )(preds_0, preds_1, preds_2, g, par)
    obox, ocls, oobj, otot = losses
    return otot[0, 0], obox[0, 0], oobj[0, 0], ocls[0, 0]


# speedup vs baseline: 17.1713x; 1.0239x over previous
import jax
import jax.numpy as jnp
from jax import lax
from jax.experimental import pallas as pl
from jax.experimental.pallas import tpu as pltpu
from jax.experimental.pallas import tpu_sc as plsc

NCLS = 80
NCHAN = 85
GAMMA = 2.0
NTGT = 4096
CH = 128
CPS = NTGT // CH
NCHUNK = 3 * CPS
NSLOT = NCHUNK * CH
NC, NS = 2, 16
NTILE = NC * NS
WSCALE = (128, 64, 32)
HWS = tuple(w * w for w in WSCALE)
CELLS = tuple(16 * hw for hw in HWS)
CELL_OFF = (0, CELLS[0], CELLS[0] + CELLS[1])
NCELL = sum(CELLS)
ZSPAN = 2048
ZPER = 11
MAPW = NS * ZPER * ZSPAN
WAVE = 17


def _sc_body(p0, p1, p2, idxh, cellh, outg,
             idx2, cell2, cellS, gbuf2, zbuf, dbuf, ones_v, cntmap,
             semg, semz, semm):
    cid = lax.axis_index("c")
    sid = lax.axis_index("s")
    wid = sid * NC + cid
    tabs = (p0, p1, p2)

    @pl.loop(0, ZSPAN // 16)
    def _(i):
        zbuf[pl.ds(i * 16, 16)] = jnp.zeros((16,), jnp.float32)

    @pl.loop(0, CH // 16)
    def _(i):
        ones_v[pl.ds(i * 16, 16)] = jnp.ones((16,), jnp.float32)

    def fire(s_const, buf):
        jj = wid + s_const * CPS
        pltpu.sync_copy(idxh.at[jj], idx2.at[buf])
        pltpu.sync_copy(cellh.at[jj], cell2.at[buf])
        p = tabs[s_const]

        @pl.loop(0, NCHAN)
        def _(c):
            pltpu.async_copy(p.at[idx2.at[buf, c]], gbuf2.at[buf, c], semg)

    def drain(s_const):
        pltpu.make_async_copy(tabs[s_const].at[pl.ds(0, NCHAN * CH)],
                              dbuf.at[pl.ds(0, NCHAN * CH)], semg).wait()

    def finish(s_const, buf):
        jj = wid + s_const * CPS
        pltpu.async_copy(cntmap.at[cell2.at[buf]], gbuf2.at[buf, NCHAN],
                         semm).wait()
        pltpu.sync_copy(gbuf2.at[buf], outg.at[jj])

    fire(0, 0)

    @pl.loop(0, ZPER)
    def _(i):
        pltpu.async_copy(zbuf,
                         cntmap.at[pl.ds((sid * ZPER + i) * ZSPAN, ZSPAN)],
                         semz)
    pltpu.make_async_copy(p0.at[pl.ds(0, ZPER * ZSPAN)], dbuf, semz).wait()
    plsc.subcore_barrier()

    @pl.loop(0, NCHUNK // NS)
    def _(k):
        jj = sid * (NCHUNK // NS) + k
        pltpu.sync_copy(cellh.at[jj], cellS)
        pltpu.sync_copy(ones_v, cntmap.at[cellS], add=True)
    plsc.subcore_barrier()

    drain(0)
    fire(1, 1)
    finish(0, 0)
    drain(1)
    fire(2, 0)
    finish(1, 1)
    drain(2)
    finish(2, 0)


def _tc_body(pl0, pl1, pl2, g, par, obox, ocls, oobj, otot):
    i = pl.program_id(0)

    def bce0(x):
        return jnp.maximum(x, 0.0) + jnp.log(1.0 + jnp.exp(-jnp.abs(x)))

    @pl.when(i == 0)
    def _():
        cxv = par[0]
        cyv = par[1]
        bwv = par[2]
        bhv = par[3]
        wgt = par[4]
        clsf = par[5]
        gxf = par[6]
        gyf = par[7]
        wf = par[8]
        hf = par[9]
        valid = par[10]
        nobj = par[11]

        x0 = g[:, 0, :]
        x1 = g[:, 1, :]
        x2 = g[:, 2, :]
        x3 = g[:, 3, :]
        x4 = g[:, 4, :]
        mult = g[:, NCHAN, :]

        px = (1.0 / (1.0 + jnp.exp(-x0)) + gxf) / wf
        py = (1.0 / (1.0 + jnp.exp(-x1)) + gyf) / hf
        pw = jnp.exp(jnp.minimum(x2, 4.0)) / wf
        ph = jnp.exp(jnp.minimum(x3, 4.0)) / hf
        l1 = (jnp.abs(px - cxv) + jnp.abs(py - cyv)
              + jnp.abs(pw - bwv) + jnp.abs(ph - bhv)) * 0.25
        box_sum = jnp.sum(jnp.where(valid > 0, l1 * wgt, 0.0))

        xc = g[:, 5:5 + NCLS, :]
        lane_c = lax.broadcasted_iota(jnp.int32, (NCHUNK, NCLS, CH), 1)
        onehot = (lane_c.astype(jnp.float32) == clsf[:, None, :]).astype(jnp.float32)
        bcec = (jnp.maximum(xc, 0.0) - xc * onehot
                + jnp.log(1.0 + jnp.exp(-jnp.abs(xc))))
        cls_sum = jnp.sum(jnp.where(valid[:, None, :] > 0, bcec, 0.0)) / NCLS

        corr = jnp.sum(jnp.where(valid > 0, x4 / mult * nobj, 0.0))

        d1 = jnp.sum(bce0(pl1[:, 0, :, :])) / (16.0 * HWS[1])
        d2 = jnp.sum(bce0(pl2[:, 0, :, :])) / (16.0 * HWS[2])

        obox[0, 0] = box_sum / NTGT
        ocls[0, 0] = cls_sum / NTGT
        oobj[0, 0] = d1 + d2 - corr

    oobj[0, 0] += jnp.sum(bce0(pl0[0, 0])) / (16.0 * HWS[0])

    @pl.when(i == 15)
    def _():
        otot[0, 0] = obox[0, 0] + ocls[0, 0] + oobj[0, 0]


def kernel(preds_0, preds_1, preds_2, targets):
    f32 = jnp.float32
    t = lax.stop_gradient(targets)
    b = t[:, 0].astype(jnp.int32)
    clsv = t[:, 1]
    cx, cy, bw, bh = t[:, 2], t[:, 3], t[:, 4], t[:, 5]
    area = jnp.maximum(bw * bh, 1e-6)
    sidx = jnp.where(area <= 0.01, 0, jnp.where(area <= 0.03, 1, 2)).astype(jnp.int32)
    weight = 1.0 + GAMMA * (1.0 - jnp.sqrt(area))

    wsa = jnp.array(WSCALE, jnp.int32).reshape(3, 1)
    hwa = jnp.array(HWS, jnp.int32).reshape(3, 1)
    offa = jnp.array(CELL_OFF, jnp.int32).reshape(3, 1)
    wsf3 = wsa.astype(f32)
    gx3 = jnp.clip((cx[None] * wsf3).astype(jnp.int32), 0, wsa - 1)
    gy3 = jnp.clip((cy[None] * wsf3).astype(jnp.int32), 0, wsa - 1)
    base3 = (b[None] * NCHAN * wsa + gy3) * 128 + gx3
    cell3 = offa + (b[None] * wsa + gy3) * wsa + gx3
    mask3 = sidx[None] == jnp.arange(3, dtype=jnp.int32)[:, None]

    gid = jnp.arange(3 * NTGT, dtype=jnp.int32).reshape(3, NTGT)
    cell3 = jnp.where(mask3, cell3, NCELL + gid % (MAPW - NCELL - 8))

    hwp = wsa * 128
    hw3 = jnp.broadcast_to(hwp, (3, NTGT))
    idx_all = (base3.reshape(NCHUNK, 1, CH)
               + jnp.arange(NCHAN, dtype=jnp.int32).reshape(1, NCHAN, 1)
               * hw3.reshape(NCHUNK, 1, CH))
    cells_arr = cell3.reshape(NCHUNK, CH)

    maskf = mask3.astype(f32)

    def brd(v):
        return jnp.broadcast_to(v[None], (3, NTGT))

    par = jnp.stack([
        brd(cx), brd(cy), brd(bw), brd(bh), brd(weight), brd(clsv),
        gx3.astype(f32), gy3.astype(f32),
        jnp.broadcast_to(wsf3, (3, NTGT)), jnp.broadcast_to(wsf3, (3, NTGT)),
        maskf,
        maskf / (16.0 * hwa.astype(f32)),
    ]).reshape(12, NCHUNK, CH)

    sc = pl.kernel(
        _sc_body,
        out_type=jax.ShapeDtypeStruct((NCHUNK, 88, CH), f32),
        mesh=plsc.VectorSubcoreMesh(core_axis_name="c", subcore_axis_name="s"),
        scratch_types=[
            pltpu.VMEM((2, NCHAN, CH), jnp.int32),
            pltpu.VMEM((2, CH), jnp.int32),
            pltpu.VMEM((CH,), jnp.int32),
            pltpu.VMEM((2, 88, CH), f32),
            pltpu.VMEM((ZSPAN,), f32),
            pltpu.VMEM((ZPER * ZSPAN,), f32),
            pltpu.VMEM((CH,), f32),
            pltpu.VMEM_SHARED((MAPW,), f32),
            pltpu.SemaphoreType.DMA,
            pltpu.SemaphoreType.DMA,
            pltpu.SemaphoreType.DMA,
        ],
    )
    p1p = jnp.pad(preds_1, ((0, 0), (0, 0), (0, 0), (0, 128 - WSCALE[1])))
    p2p = jnp.pad(preds_2, ((0, 0), (0, 0), (0, 0), (0, 128 - WSCALE[2])))
    g = sc(preds_0.reshape(-1), p1p.reshape(-1), p2p.reshape(-1),
           idx_all, cells_arr)

    losses = pl.pallas_call(
        _tc_body,
        grid=(16,),
        in_specs=[
            pl.BlockSpec((1, 1, 128, 128), lambda i: (i, 4, 0, 0)),
            pl.BlockSpec((16, 1, 64, 64), lambda i: (0, 4, 0, 0)),
            pl.BlockSpec((16, 1, 32, 32), lambda i: (0, 4, 0, 0)),
            pl.BlockSpec((NCHUNK, 88, CH), lambda i: (0, 0, 0)),
            pl.BlockSpec((12, NCHUNK, CH), lambda i: (0, 0, 0)),
        ],
        out_specs=[pl.BlockSpec((1, 1), lambda i: (0, 0),
                                memory_space=pltpu.SMEM)] * 4,
        out_shape=[jax.ShapeDtypeStruct((1, 1), f32)] * 4,
    )(preds_0, preds_1, preds_2, g, par)
    obox, ocls, oobj, otot = losses
    return otot[0, 0], obox[0, 0], oobj[0, 0], ocls[0, 0]


# speedup vs baseline: 27.3933x; 1.5953x over previous
import jax
import jax.numpy as jnp
from jax import lax
from jax.experimental import pallas as pl
from jax.experimental.pallas import tpu as pltpu
from jax.experimental.pallas import tpu_sc as plsc

NCLS = 80
NCHAN = 85
GAMMA = 2.0
NTGT = 4096
CH = 128
CPS = NTGT // CH
NCHUNK = 3 * CPS
NC, NS = 2, 16
NTILE = NC * NS
WSCALE = (128, 64, 32)
HWS = tuple(w * w for w in WSCALE)
CELLS = tuple(16 * hw for hw in HWS)
CELL_OFF = (0, CELLS[0], CELLS[0] + CELLS[1])
NCELL = sum(CELLS)
ZSPAN = 2048
ZPER = 11
MAPW = NS * ZPER * ZSPAN
SLAB = NCHAN * CH


def _sc_body(p0, t1, t2, idxh, rowh, cellh, out0, out12, outm,
             idxA, rbufA, rbufB, cellT, gbufA, slabA, slabB, mbuf,
             zbuf, dbuf, ones_v, cntmap, semg, semh, semz, semm):
    cid = lax.axis_index("c")
    sid = lax.axis_index("s")
    wid = sid * NC + cid

    @pl.loop(0, ZSPAN // 16)
    def _(i):
        zbuf[pl.ds(i * 16, 16)] = jnp.zeros((16,), jnp.float32)

    @pl.loop(0, CH // 16)
    def _(i):
        ones_v[pl.ds(i * 16, 16)] = jnp.ones((16,), jnp.float32)

    pltpu.sync_copy(idxh.at[wid], idxA)

    @pl.loop(0, NCHAN)
    def _(c):
        pltpu.async_copy(p0.at[idxA.at[c]], gbufA.at[c], semg)

    pltpu.sync_copy(rowh.at[wid], rbufA)
    pltpu.sync_copy(rowh.at[CPS + wid], rbufB)
    pltpu.async_copy(t1.at[rbufA], slabA, semh)
    pltpu.async_copy(t2.at[rbufB], slabB, semh)

    @pl.loop(0, ZPER)
    def _(i):
        pltpu.async_copy(zbuf,
                         cntmap.at[pl.ds((sid * ZPER + i) * ZSPAN, ZSPAN)],
                         semz)
    pltpu.make_async_copy(p0.at[pl.ds(0, ZPER * ZSPAN)], dbuf, semz).wait()
    plsc.subcore_barrier()

    @pl.loop(0, NCHUNK // NS)
    def _(k):
        jj = sid * (NCHUNK // NS) + k
        pltpu.sync_copy(cellh.at[jj], cellT)
        pltpu.sync_copy(ones_v, cntmap.at[cellT], add=True)

    plsc.subcore_barrier()

    pltpu.make_async_copy(p0.at[pl.ds(0, SLAB)],
                          dbuf.at[pl.ds(0, SLAB)], semg).wait()
    pltpu.sync_copy(cellh.at[wid], cellT)
    pltpu.async_copy(cntmap.at[cellT], gbufA.at[NCHAN], semm).wait()
    pltpu.sync_copy(gbufA, out0.at[wid])

    pltpu.make_async_copy(p0.at[pl.ds(0, CH * 128)],
                          dbuf.at[pl.ds(0, CH * 128)], semh).wait()
    pltpu.make_async_copy(p0.at[pl.ds(0, CH * 128)],
                          dbuf.at[pl.ds(0, CH * 128)], semh).wait()
    for s_const, slab in ((1, slabA), (2, slabB)):
        jj = (s_const - 1) * CPS + wid
        pltpu.sync_copy(cellh.at[CPS * s_const + wid], cellT)
        pltpu.async_copy(cntmap.at[cellT], mbuf, semm).wait()
        pltpu.sync_copy(slab, out12.at[jj])
        pltpu.sync_copy(mbuf, outm.at[jj])


def _tc_body(pl0, t1v, t2v, g0, g12, gm, par, obox, ocls, oobj, otot):
    i = pl.program_id(0)

    def bce0(x):
        return jnp.maximum(x, 0.0) + jnp.log(1.0 + jnp.exp(-jnp.abs(x)))

    @pl.when(i == 0)
    def _():
        f32 = jnp.float32
        p = lambda k: par[k, 0:CPS, :]
        x0, x1, x2, x3, x4 = (g0[:, c, :] for c in range(5))
        mult = g0[:, NCHAN, :]
        px = (1.0 / (1.0 + jnp.exp(-x0)) + p(6)) / p(8)
        py = (1.0 / (1.0 + jnp.exp(-x1)) + p(7)) / p(9)
        pw = jnp.exp(jnp.minimum(x2, 4.0)) / p(8)
        ph = jnp.exp(jnp.minimum(x3, 4.0)) / p(9)
        l1 = (jnp.abs(px - p(0)) + jnp.abs(py - p(1))
              + jnp.abs(pw - p(2)) + jnp.abs(ph - p(3))) * 0.25
        valid = p(10)
        box_sum = jnp.sum(jnp.where(valid > 0, l1 * p(4), 0.0))
        xc = g0[:, 5:5 + NCLS, :]
        lane_c = lax.broadcasted_iota(jnp.int32, (CPS, NCLS, CH), 1).astype(f32)
        onehot = (lane_c == p(5)[:, None, :]).astype(f32)
        bcec = (jnp.maximum(xc, 0.0) - xc * onehot
                + jnp.log(1.0 + jnp.exp(-jnp.abs(xc))))
        cls_sum = jnp.sum(jnp.where(valid[:, None, :] > 0, bcec, 0.0)) / NCLS
        corr = jnp.sum(jnp.where(valid > 0, x4 / mult * p(11), 0.0))

        q = lambda k: par[k, CPS:NCHUNK, :]
        y0 = g12[:, :, 0]
        y1 = g12[:, :, 1]
        y2 = g12[:, :, 2]
        y3 = g12[:, :, 3]
        y4 = g12[:, :, 4]
        qx = (1.0 / (1.0 + jnp.exp(-y0)) + q(6)) / q(8)
        qy = (1.0 / (1.0 + jnp.exp(-y1)) + q(7)) / q(9)
        qw = jnp.exp(jnp.minimum(y2, 4.0)) / q(8)
        qh = jnp.exp(jnp.minimum(y3, 4.0)) / q(9)
        l1q = (jnp.abs(qx - q(0)) + jnp.abs(qy - q(1))
               + jnp.abs(qw - q(2)) + jnp.abs(qh - q(3))) * 0.25
        validq = q(10)
        box_sum += jnp.sum(jnp.where(validq > 0, l1q * q(4), 0.0))
        yc = g12[:, :, 5:5 + NCLS]
        lane_q = lax.broadcasted_iota(
            jnp.int32, (2 * CPS, CH, NCLS), 2).astype(f32)
        onehot_q = (lane_q == q(5)[:, :, None]).astype(f32)
        bceq = (jnp.maximum(yc, 0.0) - yc * onehot_q
                + jnp.log(1.0 + jnp.exp(-jnp.abs(yc))))
        cls_sum += jnp.sum(jnp.where(validq[:, :, None] > 0, bceq, 0.0)) / NCLS
        corr += jnp.sum(jnp.where(validq > 0, y4 / gm[...] * q(11), 0.0))

        obox[0, 0] = box_sum / NTGT
        ocls[0, 0] = cls_sum / NTGT
        oobj[0, 0] = -corr

    oobj[0, 0] += (jnp.sum(bce0(pl0[0, 0])) / (16.0 * HWS[0])
                   + jnp.sum(bce0(t1v[:, 4:5])) / (16.0 * HWS[1])
                   + jnp.sum(bce0(t2v[:, 4:5])) / (16.0 * HWS[2]))

    @pl.when(i == 15)
    def _():
        otot[0, 0] = obox[0, 0] + ocls[0, 0] + oobj[0, 0]


def kernel(preds_0, preds_1, preds_2, targets):
    f32 = jnp.float32
    t = lax.stop_gradient(targets)
    b = t[:, 0].astype(jnp.int32)
    clsf = t[:, 1]
    cx, cy, bw, bh = t[:, 2], t[:, 3], t[:, 4], t[:, 5]
    area = jnp.maximum(bw * bh, 1e-6)
    sidx = jnp.where(area <= 0.01, 0, jnp.where(area <= 0.03, 1, 2)).astype(jnp.int32)
    weight = 1.0 + GAMMA * (1.0 - jnp.sqrt(area))

    wsa = jnp.array(WSCALE, jnp.int32).reshape(3, 1)
    hwa = jnp.array(HWS, jnp.int32).reshape(3, 1)
    offa = jnp.array(CELL_OFF, jnp.int32).reshape(3, 1)
    wsf3 = wsa.astype(f32)
    gx3 = jnp.clip((cx[None] * wsf3).astype(jnp.int32), 0, wsa - 1)
    gy3 = jnp.clip((cy[None] * wsf3).astype(jnp.int32), 0, wsa - 1)
    lcell3 = (b[None] * wsa + gy3) * wsa + gx3
    cell3 = offa + lcell3
    mask3 = sidx[None] == jnp.arange(3, dtype=jnp.int32)[:, None]

    gid = jnp.arange(3 * NTGT, dtype=jnp.int32).reshape(3, NTGT)
    cell3 = jnp.where(mask3, cell3, NCELL + gid % (MAPW - NCELL - 8))
    cells_arr = cell3.reshape(NCHUNK, CH)

    base0 = (b * NCHAN * 128 + gy3[0]) * 128 + gx3[0]
    idx0 = (base0.reshape(CPS, 1, CH)
            + jnp.arange(NCHAN, dtype=jnp.int32).reshape(1, NCHAN, 1) * HWS[0])
    rows = lcell3[1:].reshape(2 * CPS, CH)

    maskf = mask3.astype(f32)

    def brd(v):
        return jnp.broadcast_to(v[None], (3, NTGT))

    par = jnp.stack([
        brd(cx), brd(cy), brd(bw), brd(bh), brd(weight), brd(clsf),
        gx3.astype(f32), gy3.astype(f32),
        jnp.broadcast_to(wsf3, (3, NTGT)), jnp.broadcast_to(wsf3, (3, NTGT)),
        maskf,
        maskf / (16.0 * hwa.astype(f32)),
    ]).reshape(12, NCHUNK, CH)

    t1v = preds_1.transpose(0, 2, 3, 1).reshape(16 * HWS[1], NCHAN)
    t2v = preds_2.transpose(0, 2, 3, 1).reshape(16 * HWS[2], NCHAN)
    t1 = jnp.pad(t1v, ((0, 0), (0, 128 - NCHAN)))
    t2 = jnp.pad(t2v, ((0, 0), (0, 128 - NCHAN)))

    sc = pl.kernel(
        _sc_body,
        out_type=[
            jax.ShapeDtypeStruct((CPS, 88, CH), f32),
            jax.ShapeDtypeStruct((2 * CPS, CH, 128), f32),
            jax.ShapeDtypeStruct((2 * CPS, CH), f32),
        ],
        mesh=plsc.VectorSubcoreMesh(core_axis_name="c", subcore_axis_name="s"),
        scratch_types=[
            pltpu.VMEM((NCHAN, CH), jnp.int32),
            pltpu.VMEM((CH,), jnp.int32),
            pltpu.VMEM((CH,), jnp.int32),
            pltpu.VMEM((CH,), jnp.int32),
            pltpu.VMEM((88, CH), f32),
            pltpu.VMEM((CH, 128), f32),
            pltpu.VMEM((CH, 128), f32),
            pltpu.VMEM((CH,), f32),
            pltpu.VMEM((ZSPAN,), f32),
            pltpu.VMEM((ZPER * ZSPAN,), f32),
            pltpu.VMEM((CH,), f32),
            pltpu.VMEM_SHARED((MAPW,), f32),
            pltpu.SemaphoreType.DMA,
            pltpu.SemaphoreType.DMA,
            pltpu.SemaphoreType.DMA,
            pltpu.SemaphoreType.DMA,
        ],
    )
    g0, g12, gm = sc(preds_0.reshape(-1), t1, t2, idx0, rows, cells_arr)

    losses = pl.pallas_call(
        _tc_body,
        grid=(16,),
        in_specs=[
            pl.BlockSpec((1, 1, 128, 128), lambda i: (i, 4, 0, 0)),
            pl.BlockSpec((16 * HWS[1] // 16, NCHAN), lambda i: (i, 0)),
            pl.BlockSpec((16 * HWS[2] // 16, NCHAN), lambda i: (i, 0)),
            pl.BlockSpec((CPS, 88, CH), lambda i: (0, 0, 0)),
            pl.BlockSpec((2 * CPS, CH, 128), lambda i: (0, 0, 0)),
            pl.BlockSpec((2 * CPS, CH), lambda i: (0, 0)),
            pl.BlockSpec((12, NCHUNK, CH), lambda i: (0, 0, 0)),
        ],
        out_specs=[pl.BlockSpec((1, 1), lambda i: (0, 0),
                                memory_space=pltpu.SMEM)] * 4,
        out_shape=[jax.ShapeDtypeStruct((1, 1), f32)] * 4,
    )(preds_0, t1v, t2v, g0, g12, gm, par)
    obox, ocls, oobj, otot = losses
    return otot[0, 0], obox[0, 0], oobj[0, 0], ocls[0, 0]
